# baseline (speedup 1.0000x reference)
import functools

import numpy as np
import ml_dtypes

import jax
import jax.numpy as jnp
from jax import lax
from jax.experimental import pallas as pl
from jax.experimental.pallas import tpu as pltpu
from jax.experimental.pallas import tpu_sc as plsc

_NUM_CLASSES = 100000
_H = 4
_W = 4
_C = 16
_D = _H * _W * _C
_B = 16384

_NC, _NS = 2, 16
_NW = _NC * _NS
_BPW = _B // _NW
_CHUNK = 64
_NCH = _BPW // _CHUNK
_LANES = 16
_WPR = _D // 32


def _make_noise() -> np.ndarray:
    def rotl(x, r):
        return (x << np.uint32(r)) | (x >> np.uint32(32 - r))

    k0, k1 = np.uint32(0), np.uint32(1234)
    n = _B * _D
    old = np.seterr(over="ignore")
    try:
        i64 = np.arange(n, dtype=np.uint64)
        x0 = (i64 >> np.uint64(32)).astype(np.uint32)
        x1 = (i64 & np.uint64(0xFFFFFFFF)).astype(np.uint32)
        ks = [k0, k1, k0 ^ k1 ^ np.uint32(0x1BD11BDA)]
        rot = [[13, 15, 26, 6], [17, 29, 16, 24]]
        x0 += ks[0]
        x1 += ks[1]
        for i in range(5):
            for r in rot[i % 2]:
                x0 += x1
                x1 = rotl(x1, r)
                x1 ^= x0
            x0 += ks[(i + 1) % 3]
            x1 += ks[(i + 2) % 3] + np.uint32(i + 1)
        bits = x0 ^ x1
    finally:
        np.seterr(**old)

    floats = ((bits >> np.uint32(9)) | np.uint32(0x3F800000)).view(np.float32)
    floats = floats - np.float32(1.0)
    lo = np.nextafter(np.float32(-1), np.float32(0), dtype=np.float32)
    hi = np.float32(1.0)
    u = np.maximum(lo, (floats * (hi - lo) + lo).astype(np.float32))

    w = -np.log(((np.float32(1) - u) * (np.float32(1) + u))).astype(np.float32)
    wc = w - np.float32(2.5)
    p = np.float32(2.81022636e-08)
    for c in (3.43273939e-07, -3.5233877e-06, -4.39150654e-06, 0.00021858087,
              -0.00125372503, -0.00417768164, 0.246640727, 1.50140941):
        p = np.float32(c) + p * wc
    pc = p
    wt = np.sqrt(np.maximum(w, np.float32(5.0))).astype(np.float32) - np.float32(3.0)
    p = np.float32(-0.000200214257)
    for c in (0.000100950558, 0.00134934322, -0.00367342844, 0.00573950773,
              -0.0076224613, 0.00943887047, 1.00167406, 2.83297682):
        p = np.float32(c) + p * wt
    pt = p
    erfinv = (np.where(w < np.float32(5.0), pc, pt) * u).astype(np.float32)
    return (np.float32(np.sqrt(2.0)) * erfinv).reshape(_B, _D)


_NOISE_W = _make_noise().astype(ml_dtypes.bfloat16).view(np.int32)

_MESH = plsc.VectorSubcoreMesh(core_axis_name="c", subcore_axis_name="s")


_DW = _D // 2


@functools.partial(
    pl.kernel,
    out_type=jax.ShapeDtypeStruct((_B * _D,), jnp.float32),
    mesh=_MESH,
    compiler_params=pltpu.CompilerParams(needs_layout_passes=False),
    scratch_types=[
        pltpu.VMEM((_BPW,), jnp.int32),
        pltpu.VMEM((2, _CHUNK, _DW), jnp.float32),
        pltpu.VMEM((2, _CHUNK, _DW), jnp.float32),
        pltpu.VMEM((2, _CHUNK, _DW), jnp.int32),
        pltpu.VMEM((_CHUNK * _D,), jnp.float32),
        pltpu.VMEM((_CHUNK * _D,), jnp.float32),
        pltpu.SemaphoreType.DMA,
        pltpu.SemaphoreType.DMA,
        pltpu.SemaphoreType.DMA,
        pltpu.SemaphoreType.DMA,
    ],
)
def _sc_embed(labels_hbm, means_hbm, stds_hbm, noise_hbm, out_hbm,
              idx_v, mean_v, std_v, noise_v, acc_v0, acc_v1,
              sem_in0, sem_in1, sem_out0, sem_out1):
    wid = lax.axis_index("s") * _NC + lax.axis_index("c")
    base = wid * _BPW
    pltpu.sync_copy(labels_hbm.at[pl.ds(base, _BPW)], idx_v)

    sems_in = (sem_in0, sem_in1)
    sems_out = (sem_out0, sem_out1)

    def issue_in(j):
        b = j % 2
        off = j * _CHUNK
        idx = idx_v.at[pl.ds(off, _CHUNK)]
        return (
            pltpu.async_copy(means_hbm.at[idx], mean_v.at[b], sems_in[b]),
            pltpu.async_copy(stds_hbm.at[idx], std_v.at[b], sems_in[b]),
            pltpu.async_copy(noise_hbm.at[pl.ds(base + off, _CHUNK)],
                             noise_v.at[b], sems_in[b]),
        )

    accs = (acc_v0, acc_v1)
    evens = lax.iota(jnp.int32, _LANES) * 2
    hi_mask = jnp.full((_LANES,), jnp.int32(-65536))

    in_flight = {0: issue_in(0)}
    out_flight = {}
    for j in range(_NCH):
        b = j % 2
        if j + 1 < _NCH:
            if j - 1 in out_flight:
                out_flight.pop(j - 1).wait()
            in_flight[j + 1] = issue_in(j + 1)
        for c in in_flight.pop(j):
            c.wait()

        def row(r, rcarry):
            rbase = r * _D
            for k in range(_DW // _LANES):
                sl = pl.ds(k * _LANES, _LANES)
                mw = plsc.bitcast(mean_v[b, r, sl], jnp.int32)
                sw = plsc.bitcast(std_v[b, r, sl], jnp.int32)
                nw = noise_v[b, r, sl]
                me = plsc.bitcast(mw << 16, jnp.float32)
                mo = plsc.bitcast(mw & hi_mask, jnp.float32)
                se = plsc.bitcast(sw << 16, jnp.float32)
                so = plsc.bitcast(sw & hi_mask, jnp.float32)
                ne = plsc.bitcast(nw << 16, jnp.float32)
                no = plsc.bitcast(nw & hi_mask, jnp.float32)
                idx_e = evens + (rbase + k * 32)
                plsc.store_scatter(accs[b], [idx_e], me + se * ne)
                plsc.store_scatter(accs[b], [idx_e + 1], mo + so * no)
            return rcarry
        lax.fori_loop(0, _CHUNK, row, 0)

        out_flight[j] = pltpu.async_copy(
            accs[b],
            out_hbm.at[pl.ds((base + j * _CHUNK) * _D, _CHUNK * _D)],
            sems_out[b])
    for c in out_flight.values():
        c.wait()


def _pack_words(x):
    iv = lax.bitcast_convert_type(x, jnp.uint32)
    r = (iv + jnp.uint32(0x7FFF) + ((iv >> 16) & jnp.uint32(1))) >> 16
    lo = r[:, 0::2]
    hi = r[:, 1::2]
    return lax.bitcast_convert_type(lo | (hi << 16), jnp.float32)


def kernel(labels, class_means, class_stds):
    labels = labels.astype(jnp.int32)
    means = _pack_words(class_means.reshape(_NUM_CLASSES, _D))
    stds = _pack_words(class_stds.reshape(_NUM_CLASSES, _D))
    out = _sc_embed(labels, means, stds, jnp.asarray(_NOISE_W))
    return out.reshape(_B, _H, _W, _C)


# speedup vs baseline: 6.1468x; 6.1468x over previous
import functools

import numpy as np

import jax
import jax.numpy as jnp
from jax import lax
from jax.experimental import pallas as pl
from jax.experimental.pallas import tpu as pltpu
from jax.experimental.pallas import tpu_sc as plsc

_NUM_CLASSES = 100000
_H = 4
_W = 4
_C = 16
_D = _H * _W * _C
_B = 16384

_NC, _NS = 2, 16
_NW = _NC * _NS
_BPW = _B // _NW
_CHUNK = 64
_NCH = _BPW // _CHUNK
_LANES = 16


def _make_noise() -> np.ndarray:
    def rotl(x, r):
        return (x << np.uint32(r)) | (x >> np.uint32(32 - r))

    k0, k1 = np.uint32(0), np.uint32(1234)
    n = _B * _D
    old = np.seterr(over="ignore")
    try:
        i64 = np.arange(n, dtype=np.uint64)
        x0 = (i64 >> np.uint64(32)).astype(np.uint32)
        x1 = (i64 & np.uint64(0xFFFFFFFF)).astype(np.uint32)
        ks = [k0, k1, k0 ^ k1 ^ np.uint32(0x1BD11BDA)]
        rot = [[13, 15, 26, 6], [17, 29, 16, 24]]
        x0 += ks[0]
        x1 += ks[1]
        for i in range(5):
            for r in rot[i % 2]:
                x0 += x1
                x1 = rotl(x1, r)
                x1 ^= x0
            x0 += ks[(i + 1) % 3]
            x1 += ks[(i + 2) % 3] + np.uint32(i + 1)
        bits = x0 ^ x1
    finally:
        np.seterr(**old)

    floats = ((bits >> np.uint32(9)) | np.uint32(0x3F800000)).view(np.float32)
    floats = floats - np.float32(1.0)
    lo = np.nextafter(np.float32(-1), np.float32(0), dtype=np.float32)
    hi = np.float32(1.0)
    u = np.maximum(lo, (floats * (hi - lo) + lo).astype(np.float32))

    w = -np.log(((np.float32(1) - u) * (np.float32(1) + u))).astype(np.float32)
    wc = w - np.float32(2.5)
    p = np.float32(2.81022636e-08)
    for c in (3.43273939e-07, -3.5233877e-06, -4.39150654e-06, 0.00021858087,
              -0.00125372503, -0.00417768164, 0.246640727, 1.50140941):
        p = np.float32(c) + p * wc
    pc = p
    wt = np.sqrt(np.maximum(w, np.float32(5.0))).astype(np.float32) - np.float32(3.0)
    p = np.float32(-0.000200214257)
    for c in (0.000100950558, 0.00134934322, -0.00367342844, 0.00573950773,
              -0.0076224613, 0.00943887047, 1.00167406, 2.83297682):
        p = np.float32(c) + p * wt
    pt = p
    erfinv = (np.where(w < np.float32(5.0), pc, pt) * u).astype(np.float32)
    return (np.float32(np.sqrt(2.0)) * erfinv).reshape(_B, _H, _W, _C)


_NOISE = _make_noise()

_MESH = plsc.VectorSubcoreMesh(core_axis_name="c", subcore_axis_name="s")


@functools.partial(
    pl.kernel,
    out_type=jax.ShapeDtypeStruct((_B, _D), jnp.float32),
    mesh=_MESH,
    scratch_types=[
        pltpu.VMEM((_BPW,), jnp.int32),
        pltpu.VMEM((2, _CHUNK, _D), jnp.float32),
        pltpu.VMEM((2, _CHUNK, _D), jnp.float32),
        pltpu.VMEM((2, _CHUNK, _D), jnp.float32),
        pltpu.SemaphoreType.DMA,
        pltpu.SemaphoreType.DMA,
        pltpu.SemaphoreType.DMA,
        pltpu.SemaphoreType.DMA,
    ],
)
def _sc_embed(labels_hbm, means_hbm, stds_hbm, noise_hbm, out_hbm,
              idx_v, mean_v, std_v, noise_v, sem_in0, sem_in1, sem_out0, sem_out1):
    wid = lax.axis_index("s") * _NC + lax.axis_index("c")
    base = wid * _BPW
    pltpu.sync_copy(labels_hbm.at[pl.ds(base, _BPW)], idx_v)

    sems_in = (sem_in0, sem_in1)
    sems_out = (sem_out0, sem_out1)

    def issue_in(j):
        b = j % 2
        off = j * _CHUNK
        idx = idx_v.at[pl.ds(off, _CHUNK)]
        return (
            pltpu.async_copy(means_hbm.at[idx], mean_v.at[b], sems_in[b]),
            pltpu.async_copy(stds_hbm.at[idx], std_v.at[b], sems_in[b]),
            pltpu.async_copy(noise_hbm.at[pl.ds(base + off, _CHUNK)],
                             noise_v.at[b], sems_in[b]),
        )

    in_flight = {0: issue_in(0)}
    out_flight = {}
    for j in range(_NCH):
        b = j % 2
        if j + 1 < _NCH:
            nb = (j + 1) % 2
            if j - 1 in out_flight:
                out_flight.pop(j - 1).wait()
            in_flight[j + 1] = issue_in(j + 1)
        for c in in_flight.pop(j):
            c.wait()

        def row(r, rcarry):
            for c0 in range(_D // _LANES):
                sl = pl.ds(c0 * _LANES, _LANES)
                mean_v[b, r, sl] = (mean_v[b, r, sl]
                                    + std_v[b, r, sl] * noise_v[b, r, sl])
            return rcarry
        lax.fori_loop(0, _CHUNK, row, 0)

        out_flight[j] = pltpu.async_copy(
            mean_v.at[b], out_hbm.at[pl.ds(base + j * _CHUNK, _CHUNK)],
            sems_out[b])
    for c in out_flight.values():
        c.wait()


def kernel(labels, class_means, class_stds):
    labels = labels.astype(jnp.int32)
    means = class_means.reshape(_NUM_CLASSES, _D)
    stds = class_stds.reshape(_NUM_CLASSES, _D)
    out = _sc_embed(labels, means, stds, jnp.asarray(_NOISE).reshape(_B, _D))
    return out.reshape(_B, _H, _W, _C)


# speedup vs baseline: 6.1498x; 1.0005x over previous
import functools

import numpy as np

import jax
import jax.numpy as jnp
from jax import lax
from jax.experimental import pallas as pl
from jax.experimental.pallas import tpu as pltpu
from jax.experimental.pallas import tpu_sc as plsc

_NUM_CLASSES = 100000
_H = 4
_W = 4
_C = 16
_D = _H * _W * _C
_B = 16384

_NC, _NS = 2, 16
_NW = _NC * _NS
_BPW = _B // _NW
_CHUNK = 64
_NCH = _BPW // _CHUNK
_LANES = 16


def _make_noise() -> np.ndarray:
    def rotl(x, r):
        return (x << np.uint32(r)) | (x >> np.uint32(32 - r))

    k0, k1 = np.uint32(0), np.uint32(1234)
    n = _B * _D
    old = np.seterr(over="ignore")
    try:
        i64 = np.arange(n, dtype=np.uint64)
        x0 = (i64 >> np.uint64(32)).astype(np.uint32)
        x1 = (i64 & np.uint64(0xFFFFFFFF)).astype(np.uint32)
        ks = [k0, k1, k0 ^ k1 ^ np.uint32(0x1BD11BDA)]
        rot = [[13, 15, 26, 6], [17, 29, 16, 24]]
        x0 += ks[0]
        x1 += ks[1]
        for i in range(5):
            for r in rot[i % 2]:
                x0 += x1
                x1 = rotl(x1, r)
                x1 ^= x0
            x0 += ks[(i + 1) % 3]
            x1 += ks[(i + 2) % 3] + np.uint32(i + 1)
        bits = x0 ^ x1
    finally:
        np.seterr(**old)

    floats = ((bits >> np.uint32(9)) | np.uint32(0x3F800000)).view(np.float32)
    floats = floats - np.float32(1.0)
    lo = np.nextafter(np.float32(-1), np.float32(0), dtype=np.float32)
    hi = np.float32(1.0)
    u = np.maximum(lo, (floats * (hi - lo) + lo).astype(np.float32))

    w = -np.log(((np.float32(1) - u) * (np.float32(1) + u))).astype(np.float32)
    wc = w - np.float32(2.5)
    p = np.float32(2.81022636e-08)
    for c in (3.43273939e-07, -3.5233877e-06, -4.39150654e-06, 0.00021858087,
              -0.00125372503, -0.00417768164, 0.246640727, 1.50140941):
        p = np.float32(c) + p * wc
    pc = p
    wt = np.sqrt(np.maximum(w, np.float32(5.0))).astype(np.float32) - np.float32(3.0)
    p = np.float32(-0.000200214257)
    for c in (0.000100950558, 0.00134934322, -0.00367342844, 0.00573950773,
              -0.0076224613, 0.00943887047, 1.00167406, 2.83297682):
        p = np.float32(c) + p * wt
    pt = p
    erfinv = (np.where(w < np.float32(5.0), pc, pt) * u).astype(np.float32)
    return (np.float32(np.sqrt(2.0)) * erfinv).reshape(_B, _H, _W, _C)


_NOISE = _make_noise()

_MESH = plsc.VectorSubcoreMesh(core_axis_name="c", subcore_axis_name="s")


@functools.partial(
    pl.kernel,
    out_type=jax.ShapeDtypeStruct((_B, _D), jnp.float32),
    mesh=_MESH,
    scratch_types=[
        pltpu.VMEM((_BPW,), jnp.int32),
        pltpu.VMEM((2, _CHUNK, _D), jnp.float32),
        pltpu.VMEM((2, _CHUNK, _D), jnp.float32),
        pltpu.VMEM((2, _CHUNK, _D), jnp.float32),
        pltpu.SemaphoreType.DMA,
        pltpu.SemaphoreType.DMA,
        pltpu.SemaphoreType.DMA,
        pltpu.SemaphoreType.DMA,
    ],
)
def _sc_embed(labels_hbm, means_hbm, stds_hbm, noise_hbm, out_hbm,
              idx_v, mean_v, std_v, noise_v, sem_in0, sem_in1, sem_out0, sem_out1):
    wid = lax.axis_index("s") * _NC + lax.axis_index("c")
    base = wid * _BPW
    pltpu.sync_copy(labels_hbm.at[pl.ds(base, _BPW)], idx_v)

    sems_in = (sem_in0, sem_in1)
    sems_out = (sem_out0, sem_out1)

    def issue_in(j):
        b = j % 2
        off = j * _CHUNK
        idx = idx_v.at[pl.ds(off, _CHUNK)]
        return (
            pltpu.async_copy(means_hbm.at[idx], mean_v.at[b], sems_in[b]),
            pltpu.async_copy(stds_hbm.at[idx], std_v.at[b], sems_in[b]),
            pltpu.async_copy(noise_hbm.at[pl.ds(base + off, _CHUNK)],
                             noise_v.at[b], sems_in[b]),
        )

    in_flight = {0: issue_in(0)}
    out_flight = {}
    for j in range(_NCH):
        b = j % 2
        if j + 1 < _NCH:
            if j - 1 in out_flight:
                out_flight.pop(j - 1).wait()
            in_flight[j + 1] = issue_in(j + 1)
        for c in in_flight.pop(j):
            c.wait()

        def row(r, rcarry):
            for c0 in range(_D // _LANES):
                sl = pl.ds(c0 * _LANES, _LANES)
                mean_v[b, r, sl] = (mean_v[b, r, sl]
                                    + std_v[b, r, sl] * noise_v[b, r, sl])
            return rcarry
        lax.fori_loop(0, _CHUNK, row, 0)

        out_flight[j] = pltpu.async_copy(
            mean_v.at[b], out_hbm.at[pl.ds(base + j * _CHUNK, _CHUNK)],
            sems_out[b])
    for c in out_flight.values():
        c.wait()


def kernel(labels, class_means, class_stds):
    labels = labels.astype(jnp.int32)
    means = class_means.reshape(_NUM_CLASSES, _D)
    stds = class_stds.reshape(_NUM_CLASSES, _D)
    out = _sc_embed(labels, means, stds, jnp.asarray(_NOISE).reshape(_B, _D))
    return out.reshape(_B, _H, _W, _C)
